# baseline (speedup 1.0000x reference)
import jax
import jax.numpy as jnp
from jax import lax
from jax.experimental import pallas as pl
from jax.experimental.pallas import tpu as pltpu
from jax.experimental.pallas import tpu_sc as plsc

D = 64
NSEG = 16
SEQ = 512
NC = 2
NS = 16
NW = NC * NS
CHUNK = 128
NSTREAM = CHUNK // 128
LANES = 16
NBUF = 4


def _body(ids_hbm, segs_hbm, tok_hbm, comb_hbm, out_hbm,
          ids_v, cidx_v, tok_v, comb_v,
          sem_tok, sem_comb, sem_out):
    c = lax.axis_index("c")
    s = lax.axis_index("s")
    wid = c * NS + s
    batch, seq = ids_hbm.shape
    d_dim = tok_hbm.shape[1]
    n_tokens = batch * seq
    per_w = n_tokens // NW
    nchunk = per_w // CHUNK
    rows_w = per_w // SEQ
    iota = lax.iota(jnp.int32, LANES)
    wbase = wid * per_w
    wrow = wid * rows_w

    for r in range(rows_w):
        pltpu.sync_copy(ids_hbm.at[wrow + r], ids_v.at[pl.ds(r * SEQ, SEQ)])
        pltpu.sync_copy(segs_hbm.at[wrow + r], cidx_v.at[pl.ds(r * SEQ, SEQ)])

    def idx_group(g, carry):
        sl = pl.ds(g * LANES, LANES)
        cidx_v[sl] = (cidx_v[sl] * SEQ
                      + (lax.rem(g, SEQ // LANES) * LANES) + iota)
        return carry

    lax.fori_loop(0, per_w // LANES, idx_group, 0)

    def gather_descs(b, i):
        local = i * CHUNK
        descs = []
        for j in range(NSTREAM):
            sl = pl.ds(local + j * 128, 128)
            dst_sl = pl.ds(j * 128, 128)
            descs.append(pltpu.make_async_copy(
                tok_hbm.at[ids_v.at[sl]], tok_v[b].at[dst_sl], sem_tok[b]))
            descs.append(pltpu.make_async_copy(
                comb_hbm.at[cidx_v.at[sl]], comb_v[b].at[dst_sl],
                sem_comb[b]))
        return descs

    def out_slot(i):
        base = wbase + i * CHUNK
        return out_hbm.at[base // SEQ, pl.ds(lax.rem(base, SEQ), CHUNK)]

    def start(i, b):
        @pl.when(i >= NBUF)
        def _():
            pltpu.make_async_copy(tok_v[b], out_slot(i), sem_out[b]).wait()

        for d in gather_descs(b, i):
            d.start()

    def finish(i, b):
        for d in gather_descs(b, i):
            d.wait()

        def add_row(r, carry):
            for j in range(d_dim // LANES):
                sl = pl.ds(j * LANES, LANES)
                tok_v[b][r, sl] = tok_v[b][r, sl] + comb_v[b][r, sl]
            return carry

        lax.fori_loop(0, CHUNK, add_row, 0)
        pltpu.async_copy(tok_v[b], out_slot(i), sem_out[b])

    for b in range(NBUF):
        start(b, b)

    def pair_step(g, carry):
        for b in range(NBUF):
            i = g * NBUF + b
            finish(i, b)

            @pl.when(i + NBUF < nchunk)
            def _():
                start(i + NBUF, b)
        return carry

    lax.fori_loop(0, nchunk // NBUF, pair_step, 0)
    for b in range(NBUF):
        pltpu.make_async_copy(tok_v[b], out_slot(0), sem_out[b]).wait()


def kernel(input_ids, segment_ids, token_embedding_matrix,
           segment_embedding_matrix, position_embedding_matrix):
    batch, seq = input_ids.shape
    comb = (segment_embedding_matrix.astype(jnp.float32)[:, None, :]
            + position_embedding_matrix.astype(jnp.float32)[None, :, :]
            ).reshape(NSEG * SEQ, D)
    per_w = batch * seq // NW
    ids = input_ids.astype(jnp.int32)
    segs = segment_ids.astype(jnp.int32)
    tokf = token_embedding_matrix.astype(jnp.float32)

    mesh = plsc.VectorSubcoreMesh(core_axis_name="c", subcore_axis_name="s",
                                  num_cores=NC, num_subcores=NS)
    dh = D // 2
    halves = []
    for h in range(2):
        run = pl.kernel(
            _body,
            out_type=jax.ShapeDtypeStruct((batch, seq, dh), jnp.float32),
            mesh=mesh,
            compiler_params=pltpu.CompilerParams(use_tc_tiling_on_sc=False),
            scratch_types=(
                pltpu.VMEM((per_w,), jnp.int32),
                pltpu.VMEM((per_w,), jnp.int32),
                [pltpu.VMEM((CHUNK, dh), jnp.float32)] * NBUF,
                [pltpu.VMEM((CHUNK, dh), jnp.float32)] * NBUF,
                [pltpu.SemaphoreType.DMA] * NBUF,
                [pltpu.SemaphoreType.DMA] * NBUF,
                [pltpu.SemaphoreType.DMA] * NBUF,
            ),
            name=f"embed_half{h}",
        )
        halves.append(run(ids, segs, tokf[:, h * dh:(h + 1) * dh],
                          comb[:, h * dh:(h + 1) * dh]))
    return jnp.concatenate(halves, axis=-1)


# speedup vs baseline: 1.9981x; 1.9981x over previous
import jax
import jax.numpy as jnp
from jax import lax
from jax.experimental import pallas as pl
from jax.experimental.pallas import tpu as pltpu
from jax.experimental.pallas import tpu_sc as plsc

D = 64
NSEG = 16
SEQ = 512
NC = 2
NS = 16
NW = NC * NS
CHUNK = 128
NSTREAM = CHUNK // 128
LANES = 16
NBUF = 4


def _body(ids_hbm, segs_hbm, tok_hbm, comb_hbm, out_hbm,
          ids_v, cidx_v, tok_v, comb_v,
          sem_tok, sem_comb, sem_out):
    c = lax.axis_index("c")
    s = lax.axis_index("s")
    wid = c * NS + s
    batch, seq = ids_hbm.shape
    n_tokens = batch * seq
    per_w = n_tokens // NW
    nchunk = per_w // CHUNK
    rows_w = per_w // SEQ
    iota = lax.iota(jnp.int32, LANES)
    wbase = wid * per_w
    wrow = wid * rows_w

    for r in range(per_w // SEQ):
        pltpu.sync_copy(ids_hbm.at[wrow + r], ids_v.at[pl.ds(r * SEQ, SEQ)])
        pltpu.sync_copy(segs_hbm.at[wrow + r], cidx_v.at[pl.ds(r * SEQ, SEQ)])

    def idx_group(g, carry):
        sl = pl.ds(g * LANES, LANES)
        cidx_v[sl] = (cidx_v[sl] * SEQ
                      + (lax.rem(g, SEQ // LANES) * LANES) + iota)
        return carry

    lax.fori_loop(0, per_w // LANES, idx_group, 0)

    def gather_descs(b, i):
        local = i * CHUNK
        descs = []
        for j in range(NSTREAM):
            sl = pl.ds(local + j * 128, 128)
            dst_sl = pl.ds(j * 128, 128)
            descs.append(pltpu.make_async_copy(
                tok_hbm.at[ids_v.at[sl]], tok_v[b].at[dst_sl], sem_tok[b]))
            descs.append(pltpu.make_async_copy(
                comb_hbm.at[cidx_v.at[sl]], comb_v[b].at[dst_sl],
                sem_comb[b]))
        return descs

    def out_slot(i):
        base = wbase + i * CHUNK
        return out_hbm.at[base // SEQ, pl.ds(lax.rem(base, SEQ), CHUNK)]

    def start(i, b):
        @pl.when(i >= NBUF)
        def _():
            pltpu.make_async_copy(tok_v[b], out_slot(i), sem_out[b]).wait()

        for d in gather_descs(b, i):
            d.start()

    def finish(i, b):
        for d in gather_descs(b, i):
            d.wait()

        def add_row(r, carry):
            for j in range(D // LANES):
                sl = pl.ds(j * LANES, LANES)
                tok_v[b][r, sl] = tok_v[b][r, sl] + comb_v[b][r, sl]
            return carry

        lax.fori_loop(0, CHUNK, add_row, 0)
        pltpu.async_copy(tok_v[b], out_slot(i), sem_out[b])

    for b in range(NBUF):
        start(b, b)

    def pair_step(g, carry):
        for b in range(NBUF):
            i = g * NBUF + b
            finish(i, b)

            @pl.when(i + NBUF < nchunk)
            def _():
                start(i + NBUF, b)
        return carry

    lax.fori_loop(0, nchunk // NBUF, pair_step, 0)
    for b in range(NBUF):
        pltpu.make_async_copy(tok_v[b], out_slot(0), sem_out[b]).wait()


def kernel(input_ids, segment_ids, token_embedding_matrix,
           segment_embedding_matrix, position_embedding_matrix):
    batch, seq = input_ids.shape
    comb = (segment_embedding_matrix.astype(jnp.float32)[:, None, :]
            + position_embedding_matrix.astype(jnp.float32)[None, :, :]
            ).reshape(NSEG * SEQ, D)
    per_w = batch * seq // NW

    mesh = plsc.VectorSubcoreMesh(core_axis_name="c", subcore_axis_name="s",
                                  num_cores=NC, num_subcores=NS)
    run = pl.kernel(
        _body,
        out_type=jax.ShapeDtypeStruct((batch, seq, D), jnp.float32),
        mesh=mesh,
        compiler_params=pltpu.CompilerParams(use_tc_tiling_on_sc=False),
        scratch_types=(
            pltpu.VMEM((per_w,), jnp.int32),
            pltpu.VMEM((per_w,), jnp.int32),
            [pltpu.VMEM((CHUNK, D), jnp.float32)] * NBUF,
            [pltpu.VMEM((CHUNK, D), jnp.float32)] * NBUF,
            [pltpu.SemaphoreType.DMA] * NBUF,
            [pltpu.SemaphoreType.DMA] * NBUF,
            [pltpu.SemaphoreType.DMA] * NBUF,
        ),
    )
    out = run(input_ids.astype(jnp.int32), segment_ids.astype(jnp.int32),
              token_embedding_matrix.astype(jnp.float32), comb)
    return out
